# Initial kernel scaffold; baseline (speedup 1.0000x reference)
import jax
import jax.numpy as jnp
from jax.experimental import pallas as pl


def kernel(idx, idx_tem, ent_emb, tem_emb):
    raise NotImplementedError("write your pallas kernel here")


# speedup vs baseline: 1.5031x; 1.5031x over previous
import functools

import jax
import jax.numpy as jnp
from jax import lax
from jax.experimental import pallas as pl
from jax.experimental.pallas import tpu as pltpu
from jax.experimental.pallas import tpu_sc as plsc

_D = 128
_N_ENT = 10240
_N_TEM = 5120
_CHUNK = 80

_info = plsc.get_sparse_core_info()
_NC, _NS = _info.num_cores, _info.num_subcores
_NW = _NC * _NS
_ENT_CH = _N_ENT // (_NW * _CHUNK)
_TEM_CH = _N_TEM // (_NW * _CHUNK)

_mesh = plsc.VectorSubcoreMesh(core_axis_name="c", subcore_axis_name="s")


@functools.partial(
    pl.kernel,
    mesh=_mesh,
    out_type=[
        jax.ShapeDtypeStruct((_N_ENT // _CHUNK, _CHUNK, _D), jnp.float32),
        jax.ShapeDtypeStruct((_N_TEM // _CHUNK, _CHUNK, _D), jnp.float32),
    ],
    scratch_types=[
        pltpu.VMEM((_ENT_CH, _CHUNK), jnp.int32),
        pltpu.VMEM((_ENT_CH, _CHUNK, _D), jnp.float32),
        pltpu.VMEM((_TEM_CH, _CHUNK), jnp.int32),
        pltpu.VMEM((_TEM_CH, _CHUNK, _D), jnp.float32),
        pltpu.SemaphoreType.DMA,
        pltpu.SemaphoreType.DMA,
    ],
)
def _gather_kernel(ent_hbm, idx_hbm, tem_hbm, idxt_hbm, out_a, out_b,
                   idx_v, rows_v, idxt_v, rowst_v, sem_a, sem_b):
    wid = lax.axis_index("s") * _NC + lax.axis_index("c")

    ent_base = wid * _ENT_CH
    pltpu.sync_copy(idx_hbm.at[pl.ds(ent_base, _ENT_CH)], idx_v)
    ent_cps = [pltpu.async_copy(ent_hbm.at[idx_v.at[j]], rows_v.at[j], sem_a)
               for j in range(_ENT_CH)]

    tem_base = wid * _TEM_CH
    pltpu.sync_copy(idxt_hbm.at[pl.ds(tem_base, _TEM_CH)], idxt_v)
    tem_cps = [pltpu.async_copy(tem_hbm.at[idxt_v.at[j]], rowst_v.at[j], sem_b)
               for j in range(_TEM_CH)]

    for cp in ent_cps:
        cp.wait()
    pltpu.sync_copy(rows_v, out_a.at[pl.ds(ent_base, _ENT_CH)])

    for cp in tem_cps:
        cp.wait()
    pltpu.sync_copy(rowst_v, out_b.at[pl.ds(tem_base, _TEM_CH)])


def kernel(idx, idx_tem, ent_emb, tem_emb):
    B, FEW, _ = idx.shape
    idx_flat = idx.reshape(_N_ENT // _CHUNK, _CHUNK).astype(jnp.int32)
    idxt_flat = idx_tem.reshape(_N_TEM // _CHUNK, _CHUNK).astype(jnp.int32)
    a, b = _gather_kernel(ent_emb, idx_flat, tem_emb, idxt_flat)
    return (a.reshape(B, FEW, 2, _D), b.reshape(B, FEW, _D))


# speedup vs baseline: 1.5178x; 1.0098x over previous
import functools

import jax
import jax.numpy as jnp
from jax import lax
from jax.experimental import pallas as pl
from jax.experimental.pallas import tpu as pltpu
from jax.experimental.pallas import tpu_sc as plsc

_D = 128
_N_ENT = 10240
_N_TEM = 5120
_CHUNK = 80

_info = plsc.get_sparse_core_info()
_NC, _NS = _info.num_cores, _info.num_subcores
_NW = _NC * _NS
_ENT_W = _N_ENT // _NW
_TEM_W = _N_TEM // _NW
_ENT_CH = _ENT_W // _CHUNK
_TEM_CH = _TEM_W // _CHUNK

_mesh = plsc.VectorSubcoreMesh(core_axis_name="c", subcore_axis_name="s")


@functools.partial(
    pl.kernel,
    mesh=_mesh,
    out_type=[
        jax.ShapeDtypeStruct((_N_ENT // _CHUNK, _CHUNK, _D), jnp.float32),
        jax.ShapeDtypeStruct((_N_TEM // _CHUNK, _CHUNK, _D), jnp.float32),
    ],
    scratch_types=[
        pltpu.VMEM((_ENT_W,), jnp.int32),
        pltpu.VMEM((_ENT_CH, _CHUNK, _D), jnp.float32),
        pltpu.VMEM((_TEM_W,), jnp.int32),
        pltpu.VMEM((_TEM_CH, _CHUNK, _D), jnp.float32),
        pltpu.SemaphoreType.DMA((_ENT_CH,)),
        pltpu.SemaphoreType.DMA((_TEM_CH,)),
        pltpu.SemaphoreType.DMA,
    ],
)
def _gather_kernel(ent_hbm, tem_hbm, idx_hbm, out_a, out_b,
                   idx_ve, rows_v, idx_vt, rowst_v, sems_a, sems_b, sem_w):
    wid = lax.axis_index("s") * _NC + lax.axis_index("c")

    pltpu.sync_copy(idx_hbm.at[pl.ds(wid * _ENT_W, _ENT_W)], idx_ve)
    ent_cps = [
        pltpu.async_copy(ent_hbm.at[idx_ve.at[pl.ds(j * _CHUNK, _CHUNK)]],
                         rows_v.at[j], sems_a.at[j])
        for j in range(_ENT_CH)
    ]
    pltpu.sync_copy(idx_hbm.at[pl.ds(_N_ENT + wid * _TEM_W, _TEM_W)], idx_vt)
    tem_cps = [
        pltpu.async_copy(tem_hbm.at[idx_vt.at[pl.ds(j * _CHUNK, _CHUNK)]],
                         rowst_v.at[j], sems_b.at[j])
        for j in range(_TEM_CH)
    ]

    out_cps = []
    for j in range(_ENT_CH):
        ent_cps[j].wait()
        out_cps.append(pltpu.async_copy(
            rows_v.at[j], out_a.at[wid * _ENT_CH + j], sem_w))
    for j in range(_TEM_CH):
        tem_cps[j].wait()
        out_cps.append(pltpu.async_copy(
            rowst_v.at[j], out_b.at[wid * _TEM_CH + j], sem_w))
    for cp in out_cps:
        cp.wait()


def kernel(idx, idx_tem, ent_emb, tem_emb):
    B, FEW, _ = idx.shape
    idx_all = jnp.concatenate([
        idx.reshape(-1).astype(jnp.int32),
        idx_tem.reshape(-1).astype(jnp.int32),
    ])
    a, b = _gather_kernel(ent_emb, tem_emb, idx_all)
    return (a.reshape(B, FEW, 2, _D), b.reshape(B, FEW, _D))


# speedup vs baseline: 1.5749x; 1.0376x over previous
import functools

import jax
import jax.numpy as jnp
from jax import lax
from jax.experimental import pallas as pl
from jax.experimental.pallas import tpu as pltpu
from jax.experimental.pallas import tpu_sc as plsc

_D = 128
_N_ENT = 10240
_N_TEM = 5120
_CHUNK = 80

_info = plsc.get_sparse_core_info()
_NC, _NS = _info.num_cores, _info.num_subcores
_NW = _NC * _NS
_ENT_W = _N_ENT // _NW
_TEM_W = _N_TEM // _NW
_ENT_CH = _ENT_W // _CHUNK
_TEM_CH = _TEM_W // _CHUNK

_mesh = plsc.VectorSubcoreMesh(core_axis_name="c", subcore_axis_name="s")


@functools.partial(
    pl.kernel,
    mesh=_mesh,
    out_type=[
        jax.ShapeDtypeStruct((_N_ENT, _D), jnp.float32),
        jax.ShapeDtypeStruct((_N_TEM, _D), jnp.float32),
    ],
    scratch_types=[
        pltpu.VMEM((_ENT_W,), jnp.int32),
        pltpu.VMEM((_ENT_W, _D), jnp.float32),
        pltpu.VMEM((_TEM_W,), jnp.int32),
        pltpu.VMEM((_TEM_W, _D), jnp.float32),
        pltpu.SemaphoreType.DMA,
        pltpu.SemaphoreType.DMA,
    ],
)
def _gather_kernel(ent_hbm, tem_hbm, idx_hbm, out_a, out_b,
                   idx_ve, rows_v, idx_vt, rowst_v, sem_a, sem_b):
    wid = lax.axis_index("s") * _NC + lax.axis_index("c")

    pltpu.sync_copy(idx_hbm.at[pl.ds(wid * _ENT_W, _ENT_W)], idx_ve)
    ent_cp = pltpu.async_copy(ent_hbm.at[idx_ve], rows_v, sem_a)
    pltpu.sync_copy(idx_hbm.at[pl.ds(_N_ENT + wid * _TEM_W, _TEM_W)], idx_vt)
    tem_cp = pltpu.async_copy(tem_hbm.at[idx_vt], rowst_v, sem_b)

    ent_cp.wait()
    pltpu.sync_copy(rows_v, out_a.at[pl.ds(wid * _ENT_W, _ENT_W)])
    tem_cp.wait()
    pltpu.sync_copy(rowst_v, out_b.at[pl.ds(wid * _TEM_W, _TEM_W)])


def kernel(idx, idx_tem, ent_emb, tem_emb):
    B, FEW, _ = idx.shape
    idx_all = jnp.concatenate([
        idx.reshape(-1).astype(jnp.int32),
        idx_tem.reshape(-1).astype(jnp.int32),
    ])
    a, b = _gather_kernel(ent_emb, tem_emb, idx_all)
    return (a.reshape(B, FEW, 2, _D), b.reshape(B, FEW, _D))


# speedup vs baseline: 1.5978x; 1.0145x over previous
import functools

import jax
import jax.numpy as jnp
from jax import lax
from jax.experimental import pallas as pl
from jax.experimental.pallas import tpu as pltpu
from jax.experimental.pallas import tpu_sc as plsc

_D = 128
_N_ENT = 10240
_N_TEM = 5120
_CHUNK = 80

_info = plsc.get_sparse_core_info()
_NC, _NS = _info.num_cores, _info.num_subcores
_NW = _NC * _NS
_ENT_W = _N_ENT // _NW
_TEM_W = _N_TEM // _NW
_ENT_CH = _ENT_W // _CHUNK
_TEM_CH = _TEM_W // _CHUNK

_mesh = plsc.VectorSubcoreMesh(core_axis_name="c", subcore_axis_name="s")


@functools.partial(
    pl.kernel,
    mesh=_mesh,
    out_type=[
        jax.ShapeDtypeStruct((1024, 5, 2, _D), jnp.float32),
        jax.ShapeDtypeStruct((1024, 5, _D), jnp.float32),
    ],
    scratch_types=[
        pltpu.VMEM((_ENT_W,), jnp.int32),
        pltpu.VMEM((_ENT_W, _D), jnp.float32),
        pltpu.VMEM((_TEM_W,), jnp.int32),
        pltpu.VMEM((_TEM_W, _D), jnp.float32),
        pltpu.SemaphoreType.DMA,
        pltpu.SemaphoreType.DMA,
    ],
)
def _gather_kernel(ent_hbm, tem_hbm, idx_hbm, out_a, out_b,
                   idx_ve, rows_v, idx_vt, rowst_v, sem_a, sem_b):
    wid = lax.axis_index("s") * _NC + lax.axis_index("c")

    pltpu.sync_copy(idx_hbm.at[pl.ds(wid * _ENT_W, _ENT_W)], idx_ve)
    ent_cp = pltpu.async_copy(ent_hbm.at[idx_ve], rows_v, sem_a)
    pltpu.sync_copy(idx_hbm.at[pl.ds(_N_ENT + wid * _TEM_W, _TEM_W)], idx_vt)
    tem_cp = pltpu.async_copy(tem_hbm.at[idx_vt], rowst_v, sem_b)

    ent_cp.wait()
    pltpu.sync_copy(rows_v.reshape(_ENT_W // 10, 5, 2, _D),
                    out_a.at[pl.ds(wid * (_ENT_W // 10), _ENT_W // 10)])
    tem_cp.wait()
    pltpu.sync_copy(rowst_v.reshape(_TEM_W // 5, 5, _D),
                    out_b.at[pl.ds(wid * (_TEM_W // 5), _TEM_W // 5)])


def kernel(idx, idx_tem, ent_emb, tem_emb):
    B, FEW, _ = idx.shape
    idx_all = jnp.concatenate([
        idx.reshape(-1).astype(jnp.int32),
        idx_tem.reshape(-1).astype(jnp.int32),
    ])
    a, b = _gather_kernel(ent_emb, tem_emb, idx_all)
    return (a, b)
